# Initial kernel scaffold; baseline (speedup 1.0000x reference)
import functools

import jax
import jax.numpy as jnp
from jax import lax
from jax.experimental import pallas as pl
from jax.experimental.pallas import tpu as pltpu
from jax.experimental.pallas import tpu_sc as plsc

N = 50000
E = 800000
NSC = 2
NT = 16
HALF = 25000
ACC = 25088
RPT = 1568
WPT = 392
EPT = E // NT
ICH = 2000
NICH = EPT // ICH
CAPC = 392
CH = 128
SUP = 4


def _fill_f32(ref, rows, value):
  v = jnp.full((16,), value, jnp.float32)
  def body(i, _):
    r = i >> 2
    c = (i & 3) * 16
    ref[r, pl.ds(c, 16)] = v
    return 0
  lax.fori_loop(0, rows * 4, body, 0)


def _prep_body(src_h, dst_h, deg_h, srcl_h, dstl_h, cnk_h,
               deg_sh, csrc, cdst, sbuf, dbuf, zbuf, ones_v, nbuf, sem):
  c = lax.axis_index("c")
  s = lax.axis_index("s")
  lo = c * HALF

  _fill_f32(zbuf, WPT, 0.0)
  zflat = zbuf.reshape(WPT * 64)
  for p in range(4):
    pltpu.sync_copy(zflat.at[pl.ds(0, RPT)], deg_sh.at[pl.ds(s * RPT, RPT)])
  one = jnp.full((16,), 1.0, jnp.float32)
  for g in range(8):
    ones_v[pl.ds(g * 16, 16)] = one
  plsc.subcore_barrier()

  iota = lax.iota(jnp.int32, 16)
  ebase = s * EPT

  def chunk_body(j, off):
    pltpu.sync_copy(src_h.at[pl.ds(ebase + j * ICH, ICH)], sbuf)
    pltpu.sync_copy(dst_h.at[pl.ds(ebase + j * ICH, ICH)], dbuf)

    def grp(gi, off):
      sv = sbuf[pl.ds(gi * 16, 16)]
      dv = dbuf[pl.ds(gi * 16, 16)]
      m = (dv >= lo) & (dv < lo + HALF)
      dl = dv - lo
      ones_i = jnp.where(m, 1, 0).astype(jnp.int32)
      pos = off + plsc.cumsum(ones_i) - 1
      plsc.store_scatter(csrc, [pos >> 7, pos & 127], sv, mask=m)
      plsc.store_scatter(cdst, [pos >> 7, pos & 127], dl, mask=m)
      return off + plsc.all_reduce_population_count(m)

    return lax.fori_loop(0, ICH // 16, grp, off)

  off = lax.fori_loop(0, NICH, chunk_body, jnp.zeros((16,), jnp.int32))

  pe = ((off + 511) >> 9) << 9

  def pad_grp(t, _):
    pos = off + t * 16 + iota
    m = pos < pe
    dump = HALF + (pos & 63)
    spread = (pos * 61) & 16383
    plsc.store_scatter(cdst, [pos >> 7, pos & 127], dump, mask=m)
    plsc.store_scatter(csrc, [pos >> 7, pos & 127], spread, mask=m)
    return 0
  lax.fori_loop(0, 32, pad_grp, 0)

  ncnk = pe >> 7
  n = jnp.max(ncnk, axis=0)

  def deg_body(jc, _):
    pltpu.sync_copy(ones_v, deg_sh.at[cdst.at[jc]], add=True)
    return 0
  lax.fori_loop(0, n, deg_body, 0)

  def wr_body(j8, _):
    pltpu.sync_copy(csrc.at[pl.ds(j8 * 8, 8)], srcl_h.at[c, s, pl.ds(j8 * 8, 8)])
    pltpu.sync_copy(cdst.at[pl.ds(j8 * 8, 8)], dstl_h.at[c, s, pl.ds(j8 * 8, 8)])
    return 0
  lax.fori_loop(0, (n + 7) >> 3, wr_body, 0)

  nbuf[...] = ncnk
  pltpu.sync_copy(nbuf, cnk_h.at[c, s])

  plsc.subcore_barrier()
  pltpu.sync_copy(deg_sh.at[pl.ds(s * RPT, RPT)], zflat.at[pl.ds(0, RPT)])
  pltpu.sync_copy(zflat.at[pl.ds(0, RPT)], deg_h.at[c, pl.ds(s * RPT, RPT)])


@functools.partial(
    pl.kernel,
    out_type=(
        jax.ShapeDtypeStruct((NSC, ACC), jnp.float32),
        jax.ShapeDtypeStruct((NSC, NT, CAPC, CH), jnp.int32),
        jax.ShapeDtypeStruct((NSC, NT, CAPC, CH), jnp.int32),
        jax.ShapeDtypeStruct((NSC, NT, 16), jnp.int32),
    ),
    mesh=plsc.VectorSubcoreMesh(core_axis_name="c", subcore_axis_name="s"),
    scratch_types=[
        pltpu.VMEM_SHARED((ACC,), jnp.float32),
        pltpu.VMEM((CAPC, CH), jnp.int32),
        pltpu.VMEM((CAPC, CH), jnp.int32),
        pltpu.VMEM((ICH,), jnp.int32),
        pltpu.VMEM((ICH,), jnp.int32),
        pltpu.VMEM((WPT, 64), jnp.float32),
        pltpu.VMEM((CH,), jnp.float32),
        pltpu.VMEM((16,), jnp.int32),
        pltpu.SemaphoreType.DMA,
    ],
)
def _sc_prep(src_h, dst_h, deg_h, srcl_h, dstl_h, cnk_h,
             deg_sh, csrc, cdst, sbuf, dbuf, zbuf, ones_v, nbuf, sem):
  _prep_body(src_h, dst_h, deg_h, srcl_h, dstl_h, cnk_h,
             deg_sh, csrc, cdst, sbuf, dbuf, zbuf, ones_v, nbuf, sem)


def _conv_body(g_h, srcl_h, dstl_h, cnk_h, acc_h,
               acc_sh, sidx, didx, rows, outb, nbuf, gsem):
  c = lax.axis_index("c")
  s = lax.axis_index("s")

  _fill_f32(outb, WPT, 0.0)
  for p in range(4):
    pltpu.sync_copy(outb, acc_sh.at[pl.ds(s * RPT + p * WPT, WPT)])
  plsc.subcore_barrier()

  pltpu.sync_copy(cnk_h.at[c, s], nbuf)
  n = jnp.max(nbuf[...], axis=0)

  def sup_body(j, _):
    pltpu.sync_copy(srcl_h.at[c, s, pl.ds(j * SUP, SUP)], sidx)
    pltpu.sync_copy(dstl_h.at[c, s, pl.ds(j * SUP, SUP)], didx)
    for k in range(SUP):
      pltpu.async_copy(g_h.at[sidx.at[k]], rows.at[k], gsem).wait()
    for k in range(SUP):
      pltpu.sync_copy(rows.at[k], acc_sh.at[didx.at[k]], add=True)
    return 0

  lax.fori_loop(0, n >> 2, sup_body, 0)
  plsc.subcore_barrier()

  for p in range(4):
    pltpu.sync_copy(acc_sh.at[pl.ds(s * RPT + p * WPT, WPT)], outb)
    pltpu.sync_copy(outb, acc_h.at[c, pl.ds(s * RPT + p * WPT, WPT)])


@functools.partial(
    pl.kernel,
    out_type=jax.ShapeDtypeStruct((NSC, ACC, 64), jnp.float32),
    mesh=plsc.VectorSubcoreMesh(core_axis_name="c", subcore_axis_name="s"),
    scratch_types=[
        pltpu.VMEM_SHARED((ACC, 64), jnp.float32),
        pltpu.VMEM((SUP, CH), jnp.int32),
        pltpu.VMEM((SUP, CH), jnp.int32),
        pltpu.VMEM((SUP, CH, 64), jnp.float32),
        pltpu.VMEM((WPT, 64), jnp.float32),
        pltpu.VMEM((16,), jnp.int32),
        pltpu.SemaphoreType.DMA,
    ],
)
def _sc_conv(g_h, srcl_h, dstl_h, cnk_h, acc_h,
             acc_sh, sidx, didx, rows, outb, nbuf, gsem):
  _conv_body(g_h, srcl_h, dstl_h, cnk_h, acc_h,
             acc_sh, sidx, didx, rows, outb, nbuf, gsem)


BN = 2000


def _k1_body(xb_r, xr_r, xt_r, deg_r, wn_r, bn_r, wr_r, br_r, wt_r, bt_r,
             wf_r, bf_r, wg1_r, g1_r):
  f32 = jnp.float32
  hb = jax.nn.relu(jnp.dot(xb_r[...], wn_r[...], preferred_element_type=f32) + bn_r[...])
  hr = jax.nn.relu(jnp.dot(xr_r[...], wr_r[...], preferred_element_type=f32) + br_r[...])
  ht = jax.nn.relu(jnp.dot(xt_r[...], wt_r[...], preferred_element_type=f32) + bt_r[...])
  wf = wf_r[...]
  pre = (jnp.dot(hb, wf[0:64], preferred_element_type=f32)
         + jnp.dot(hr, wf[64:128], preferred_element_type=f32)
         + jnp.dot(ht, wf[128:160], preferred_element_type=f32)
         + bf_r[...])
  h = jnp.where(pre >= 0, pre, 0.01 * pre)
  hw = jnp.dot(h, wg1_r[...], preferred_element_type=f32)
  dinv = lax.rsqrt(deg_r[...] + 1.0)
  g1_r[...] = hw * dinv


def _tc_k1(x_bbox, x_rf, x_txp, deg, Wn, bn, Wr, br, Wt, bt, Wf, bf, Wg1):
  full = lambda shape: pl.BlockSpec(shape, lambda i: (0, 0))
  row = lambda d: pl.BlockSpec((BN, d), lambda i: (i, 0))
  return pl.pallas_call(
      _k1_body,
      grid=(N // BN,),
      in_specs=[row(32), row(64), row(32), row(1),
                full((32, 64)), full((1, 64)), full((64, 64)), full((1, 64)),
                full((32, 32)), full((1, 32)), full((160, 128)), full((1, 128)),
                full((128, 64))],
      out_specs=row(64),
      out_shape=jax.ShapeDtypeStruct((N, 64), jnp.float32),
  )(x_bbox, x_rf, x_txp, deg, Wn, bn.reshape(1, -1), Wr, br.reshape(1, -1),
    Wt, bt.reshape(1, -1), Wf, bf.reshape(1, -1), Wg1)


def _k2_body(acc_r, g1_r, deg_r, bg1_r, wg2_r, h1_r, g2_r):
  f32 = jnp.float32
  dinv = lax.rsqrt(deg_r[...] + 1.0)
  h1 = jax.nn.relu(dinv * (acc_r[...] + g1_r[...]) + bg1_r[...])
  h1_r[...] = h1
  g2_r[...] = jnp.dot(h1, wg2_r[...], preferred_element_type=f32) * dinv


def _tc_k2(acc1, g1, deg, bg1, Wg2):
  full = lambda shape: pl.BlockSpec(shape, lambda i: (0, 0))
  row = lambda d: pl.BlockSpec((BN, d), lambda i: (i, 0))
  return pl.pallas_call(
      _k2_body,
      grid=(N // BN,),
      in_specs=[row(64), row(64), row(1), full((1, 64)), full((64, 64))],
      out_specs=[row(64), row(64)],
      out_shape=[jax.ShapeDtypeStruct((N, 64), jnp.float32),
                 jax.ShapeDtypeStruct((N, 64), jnp.float32)],
  )(acc1, g1, deg, bg1.reshape(1, -1), Wg2)


def _k3_body(acc_r, g2_r, deg_r, h1_r, bg2_r, ga_r, be_r, wo_r, bo_r, out_r):
  f32 = jnp.float32
  dinv = lax.rsqrt(deg_r[...] + 1.0)
  h2 = jax.nn.relu(dinv * (acc_r[...] + g2_r[...]) + bg2_r[...])
  scale = 1.0 / jnp.sqrt(jnp.float32(1.0 + 1e-5))
  ga = ga_r[...] * scale
  be = be_r[...]
  wo = wo_r[...]
  x1 = h1_r[...] * ga[:, 0:64] + be[:, 0:64]
  x2 = h2 * ga[:, 64:128] + be[:, 64:128]
  pre = (jnp.dot(x1, wo[0:64], preferred_element_type=f32)
         + jnp.dot(x2, wo[64:128], preferred_element_type=f32)
         + bo_r[...])
  out_r[...] = jax.nn.relu(pre)


def _tc_k3(acc2, g2, deg, h1, bg2, gamma, beta, Wo, bo):
  full = lambda shape: pl.BlockSpec(shape, lambda i: (0, 0))
  row = lambda d: pl.BlockSpec((BN, d), lambda i: (i, 0))
  return pl.pallas_call(
      _k3_body,
      grid=(N // BN,),
      in_specs=[row(64), row(64), row(1), row(64), full((1, 64)),
                full((1, 128)), full((1, 128)), full((128, 128)), full((1, 128))],
      out_specs=row(128),
      out_shape=jax.ShapeDtypeStruct((N, 128), jnp.float32),
  )(acc2, g2, deg, h1, bg2.reshape(1, -1), gamma.reshape(1, -1),
    beta.reshape(1, -1), Wo, bo.reshape(1, -1))


def _halves(x2):
  return jnp.concatenate([x2[0, :HALF], x2[1, :HALF]], axis=0)


@jax.jit
def kernel(x_bbox, x_rf, x_txp, edge_index, Wn, bn, Wr, br, Wt, bt, Wf, bf,
           Wg1, bg1, Wg2, bg2, gamma, beta, Wo, bo):
  src = edge_index[0]
  dst = edge_index[1]
  deg2, srcl, dstl, cnk = _sc_prep(src, dst)
  deg = _halves(deg2).reshape(N, 1)
  g1 = _tc_k1(x_bbox, x_rf, x_txp, deg, Wn, bn, Wr, br, Wt, bt, Wf, bf, Wg1)
  acc1 = _halves(_sc_conv(g1, srcl, dstl, cnk))
  h1, g2 = _tc_k2(acc1, g1, deg, bg1, Wg2)
  acc2 = _halves(_sc_conv(g2, srcl, dstl, cnk))
  return _tc_k3(acc2, g2, deg, h1, bg2, gamma, beta, Wo, bo)


# speedup vs baseline: 17.8919x; 17.8919x over previous
import functools

import jax
import jax.numpy as jnp
from jax import lax
from jax.experimental import pallas as pl
from jax.experimental.pallas import tpu as pltpu
from jax.experimental.pallas import tpu_sc as plsc

N = 50000
E = 800000
NSC = 2
NT = 16
HALF = 25000
ACC = 25088
RPT = 1568
WPT = 112
EPT = E // NT
ICH = 2000
NICH = EPT // ICH
CAPC = 392
CH = 128
SUP = 8


def _m8(x):
  return pl.multiple_of(x, 8)


_GDN = lax.GatherDimensionNumbers(
    offset_dims=(), collapsed_slice_dims=(0,), start_index_map=(0,))


def _prefix16(x, iota):
  for sh in (1, 2, 4, 8):
    idx = jnp.maximum(iota - sh, 0)
    shifted = lax.gather(x, idx[:, None], _GDN, slice_sizes=(1,),
                         mode=lax.GatherScatterMode.PROMISE_IN_BOUNDS)
    x = x + jnp.where(iota >= sh, shifted, 0)
  return x


def _fill_f32(ref, rows, value):
  v = jnp.full((16,), value, jnp.float32)
  def body(i, _):
    r = i >> 2
    c = (i & 3) * 16
    ref[r, pl.ds(c, 16)] = v
    return 0
  lax.fori_loop(0, rows * 4, body, 0)


def _prep_body(src_h, dst_h, deg_h, srcl_h, dstl_h, cnk_h,
               deg_sh, csrc, cdst, sbuf, dbuf, zbuf, ones_v, nbuf, sem):
  c = lax.axis_index("c")
  s = lax.axis_index("s")
  lo = c * HALF

  zero = jnp.zeros((16,), jnp.float32)
  def zfill(i, _):
    zbuf[pl.ds(i * 16, 16)] = zero
    return 0
  lax.fori_loop(0, RPT // 16, zfill, 0)
  pltpu.sync_copy(zbuf, deg_sh.at[pl.ds(_m8(s * RPT), RPT)])
  one = jnp.full((16,), 1.0, jnp.float32)
  for g in range(8):
    ones_v[pl.ds(g * 16, 16)] = one
  plsc.subcore_barrier()

  iota = lax.iota(jnp.int32, 16)
  ebase = s * EPT

  def chunk_body(j, off):
    pltpu.sync_copy(src_h.at[pl.ds(_m8(ebase + j * ICH), ICH)], sbuf)
    pltpu.sync_copy(dst_h.at[pl.ds(_m8(ebase + j * ICH), ICH)], dbuf)

    def grp(gi, off):
      sv = sbuf[pl.ds(gi * 16, 16)]
      dv = dbuf[pl.ds(gi * 16, 16)]
      m = (dv >= lo) & (dv < lo + HALF)
      dl = dv - lo
      pref = _prefix16(jnp.where(m, 1, 0).astype(jnp.int32), iota)
      pos = off + pref - 1
      plsc.store_scatter(csrc, [pos >> 7, pos & 127], sv, mask=m)
      plsc.store_scatter(cdst, [pos >> 7, pos & 127], dl, mask=m)
      return off + pref[15]

    return lax.fori_loop(0, ICH // 16, grp, off)

  off = lax.fori_loop(0, NICH, chunk_body, jnp.int32(0))

  pe = ((off + 1023) >> 10) << 10

  def pad_grp(t, _):
    pos = off + t * 16 + iota
    m = pos < pe
    dump = HALF + (pos & 63)
    spread = (pos * 61) & 16383
    plsc.store_scatter(cdst, [pos >> 7, pos & 127], dump, mask=m)
    plsc.store_scatter(csrc, [pos >> 7, pos & 127], spread, mask=m)
    return 0
  lax.fori_loop(0, 64, pad_grp, 0)

  n = pe >> 7

  def deg_body(jc, _):
    pltpu.sync_copy(ones_v, deg_sh.at[cdst.at[jc]], add=True)
    return 0
  lax.fori_loop(0, n, deg_body, 0)

  def wr_body(j8, _):
    pltpu.sync_copy(csrc.at[pl.ds(_m8(j8 * 8), 8)], srcl_h.at[c, s, pl.ds(_m8(j8 * 8), 8)])
    pltpu.sync_copy(cdst.at[pl.ds(_m8(j8 * 8), 8)], dstl_h.at[c, s, pl.ds(_m8(j8 * 8), 8)])
    return 0
  lax.fori_loop(0, (n + 7) >> 3, wr_body, 0)

  nbuf[...] = jnp.full((16,), n, jnp.int32)
  pltpu.sync_copy(nbuf, cnk_h.at[pl.ds(_m8((c * NT + s) * 16), 16)])

  plsc.subcore_barrier()
  pltpu.sync_copy(deg_sh.at[pl.ds(_m8(s * RPT), RPT)], zbuf)
  pltpu.sync_copy(zbuf, deg_h.at[pl.ds(_m8(c * ACC + s * RPT), RPT)])


@functools.partial(
    pl.kernel,
    out_type=(
        jax.ShapeDtypeStruct((NSC * ACC,), jnp.float32),
        jax.ShapeDtypeStruct((NSC, NT, CAPC, CH), jnp.int32),
        jax.ShapeDtypeStruct((NSC, NT, CAPC, CH), jnp.int32),
        jax.ShapeDtypeStruct((NSC * NT * 16,), jnp.int32),
    ),
    mesh=plsc.VectorSubcoreMesh(core_axis_name="c", subcore_axis_name="s"),
    compiler_params=pltpu.CompilerParams(needs_layout_passes=False, use_tc_tiling_on_sc=False),
    scratch_types=[
        pltpu.VMEM_SHARED((ACC,), jnp.float32),
        pltpu.VMEM((CAPC, CH), jnp.int32),
        pltpu.VMEM((CAPC, CH), jnp.int32),
        pltpu.VMEM((ICH,), jnp.int32),
        pltpu.VMEM((ICH,), jnp.int32),
        pltpu.VMEM((RPT,), jnp.float32),
        pltpu.VMEM((CH,), jnp.float32),
        pltpu.VMEM((16,), jnp.int32),
        pltpu.SemaphoreType.DMA,
    ],
)
def _sc_prep(src_h, dst_h, deg_h, srcl_h, dstl_h, cnk_h,
             deg_sh, csrc, cdst, sbuf, dbuf, zbuf, ones_v, nbuf, sem):
  _prep_body(src_h, dst_h, deg_h, srcl_h, dstl_h, cnk_h,
             deg_sh, csrc, cdst, sbuf, dbuf, zbuf, ones_v, nbuf, sem)


def _conv_body(g_h, srcl_h, dstl_h, cnk_h, acc_h,
               acc_sh, sidx, didx, rows, outb, nbuf, gsem):
  c = lax.axis_index("c")
  s = lax.axis_index("s")

  _fill_f32(outb, WPT, 0.0)
  for p in range(RPT // WPT):
    pltpu.sync_copy(outb, acc_sh.at[pl.ds(_m8(s * RPT + p * WPT), WPT)])
  plsc.subcore_barrier()

  pltpu.sync_copy(cnk_h.at[pl.ds(_m8((c * NT + s) * 16), 16)], nbuf)
  n = nbuf[...][0]

  def sup_body(j, _):
    pltpu.sync_copy(srcl_h.at[c, s, pl.ds(_m8(j * SUP), SUP)], sidx)
    pltpu.sync_copy(dstl_h.at[c, s, pl.ds(_m8(j * SUP), SUP)], didx)
    for k in range(SUP):
      pltpu.async_copy(g_h.at[sidx.at[k]], rows.at[k & 1], gsem).wait()
      pltpu.sync_copy(rows.at[k & 1], acc_sh.at[didx.at[k]], add=True)
    return 0

  lax.fori_loop(0, n >> 3, sup_body, 0)
  plsc.subcore_barrier()

  for p in range(RPT // WPT):
    pltpu.sync_copy(acc_sh.at[pl.ds(_m8(s * RPT + p * WPT), WPT)], outb)
    pltpu.sync_copy(outb, acc_h.at[c, pl.ds(_m8(s * RPT + p * WPT), WPT)])


@functools.partial(
    pl.kernel,
    out_type=jax.ShapeDtypeStruct((NSC, ACC, 64), jnp.float32),
    mesh=plsc.VectorSubcoreMesh(core_axis_name="c", subcore_axis_name="s"),
    compiler_params=pltpu.CompilerParams(needs_layout_passes=False, use_tc_tiling_on_sc=False),
    scratch_types=[
        pltpu.VMEM_SHARED((ACC, 64), jnp.float32),
        pltpu.VMEM((SUP, CH), jnp.int32),
        pltpu.VMEM((SUP, CH), jnp.int32),
        pltpu.VMEM((2, CH, 64), jnp.float32),
        pltpu.VMEM((WPT, 64), jnp.float32),
        pltpu.VMEM((16,), jnp.int32),
        pltpu.SemaphoreType.DMA,
    ],
)
def _sc_conv(g_h, srcl_h, dstl_h, cnk_h, acc_h,
             acc_sh, sidx, didx, rows, outb, nbuf, gsem):
  _conv_body(g_h, srcl_h, dstl_h, cnk_h, acc_h,
             acc_sh, sidx, didx, rows, outb, nbuf, gsem)


BN = 2000


def _k1_body(xb_r, xr_r, xt_r, deg_r, wn_r, bn_r, wr_r, br_r, wt_r, bt_r,
             wf_r, bf_r, wg1_r, g1_r):
  f32 = jnp.float32
  hb = jax.nn.relu(jnp.dot(xb_r[...], wn_r[...], preferred_element_type=f32) + bn_r[...])
  hr = jax.nn.relu(jnp.dot(xr_r[...], wr_r[...], preferred_element_type=f32) + br_r[...])
  ht = jax.nn.relu(jnp.dot(xt_r[...], wt_r[...], preferred_element_type=f32) + bt_r[...])
  wf = wf_r[...]
  pre = (jnp.dot(hb, wf[0:64], preferred_element_type=f32)
         + jnp.dot(hr, wf[64:128], preferred_element_type=f32)
         + jnp.dot(ht, wf[128:160], preferred_element_type=f32)
         + bf_r[...])
  h = jnp.where(pre >= 0, pre, 0.01 * pre)
  hw = jnp.dot(h, wg1_r[...], preferred_element_type=f32)
  dinv = lax.rsqrt(deg_r[...] + 1.0)
  g1_r[...] = hw * dinv


def _tc_k1(x_bbox, x_rf, x_txp, deg, Wn, bn, Wr, br, Wt, bt, Wf, bf, Wg1):
  full = lambda shape: pl.BlockSpec(shape, lambda i: (0, 0))
  row = lambda d: pl.BlockSpec((BN, d), lambda i: (i, 0))
  return pl.pallas_call(
      _k1_body,
      grid=(N // BN,),
      in_specs=[row(32), row(64), row(32), row(1),
                full((32, 64)), full((1, 64)), full((64, 64)), full((1, 64)),
                full((32, 32)), full((1, 32)), full((160, 128)), full((1, 128)),
                full((128, 64))],
      out_specs=row(64),
      out_shape=jax.ShapeDtypeStruct((N, 64), jnp.float32),
  )(x_bbox, x_rf, x_txp, deg, Wn, bn.reshape(1, -1), Wr, br.reshape(1, -1),
    Wt, bt.reshape(1, -1), Wf, bf.reshape(1, -1), Wg1)


def _k2_body(acc_r, g1_r, deg_r, bg1_r, wg2_r, h1_r, g2_r):
  f32 = jnp.float32
  dinv = lax.rsqrt(deg_r[...] + 1.0)
  h1 = jax.nn.relu(dinv * (acc_r[...] + g1_r[...]) + bg1_r[...])
  h1_r[...] = h1
  g2_r[...] = jnp.dot(h1, wg2_r[...], preferred_element_type=f32) * dinv


def _tc_k2(acc1, g1, deg, bg1, Wg2):
  full = lambda shape: pl.BlockSpec(shape, lambda i: (0, 0))
  row = lambda d: pl.BlockSpec((BN, d), lambda i: (i, 0))
  return pl.pallas_call(
      _k2_body,
      grid=(N // BN,),
      in_specs=[row(64), row(64), row(1), full((1, 64)), full((64, 64))],
      out_specs=[row(64), row(64)],
      out_shape=[jax.ShapeDtypeStruct((N, 64), jnp.float32),
                 jax.ShapeDtypeStruct((N, 64), jnp.float32)],
  )(acc1, g1, deg, bg1.reshape(1, -1), Wg2)


def _k3_body(acc_r, g2_r, deg_r, h1_r, bg2_r, ga_r, be_r, wo_r, bo_r, out_r):
  f32 = jnp.float32
  dinv = lax.rsqrt(deg_r[...] + 1.0)
  h2 = jax.nn.relu(dinv * (acc_r[...] + g2_r[...]) + bg2_r[...])
  scale = 1.0 / jnp.sqrt(jnp.float32(1.0 + 1e-5))
  ga = ga_r[...] * scale
  be = be_r[...]
  wo = wo_r[...]
  x1 = h1_r[...] * ga[:, 0:64] + be[:, 0:64]
  x2 = h2 * ga[:, 64:128] + be[:, 64:128]
  pre = (jnp.dot(x1, wo[0:64], preferred_element_type=f32)
         + jnp.dot(x2, wo[64:128], preferred_element_type=f32)
         + bo_r[...])
  out_r[...] = jax.nn.relu(pre)


def _tc_k3(acc2, g2, deg, h1, bg2, gamma, beta, Wo, bo):
  full = lambda shape: pl.BlockSpec(shape, lambda i: (0, 0))
  row = lambda d: pl.BlockSpec((BN, d), lambda i: (i, 0))
  return pl.pallas_call(
      _k3_body,
      grid=(N // BN,),
      in_specs=[row(64), row(64), row(1), row(64), full((1, 64)),
                full((1, 128)), full((1, 128)), full((128, 128)), full((1, 128))],
      out_specs=row(128),
      out_shape=jax.ShapeDtypeStruct((N, 128), jnp.float32),
  )(acc2, g2, deg, h1, bg2.reshape(1, -1), gamma.reshape(1, -1),
    beta.reshape(1, -1), Wo, bo.reshape(1, -1))


def _halves(x2):
  return jnp.concatenate([x2[0, :HALF], x2[1, :HALF]], axis=0)


@jax.jit
def kernel(x_bbox, x_rf, x_txp, edge_index, Wn, bn, Wr, br, Wt, bt, Wf, bf,
           Wg1, bg1, Wg2, bg2, gamma, beta, Wo, bo):
  src = edge_index[0]
  dst = edge_index[1]
  deg2, srcl, dstl, cnk = _sc_prep(src, dst)
  deg = _halves(deg2.reshape(NSC, ACC)).reshape(N, 1)
  g1 = _tc_k1(x_bbox, x_rf, x_txp, deg, Wn, bn, Wr, br, Wt, bt, Wf, bf, Wg1)
  acc1 = _halves(_sc_conv(g1, srcl, dstl, cnk))
  h1, g2 = _tc_k2(acc1, g1, deg, bg1, Wg2)
  acc2 = _halves(_sc_conv(g2, srcl, dstl, cnk))
  return _tc_k3(acc2, g2, deg, h1, bg2, gamma, beta, Wo, bo)


# speedup vs baseline: 20.0375x; 1.1199x over previous
import functools

import jax
import jax.numpy as jnp
from jax import lax
from jax.experimental import pallas as pl
from jax.experimental.pallas import tpu as pltpu
from jax.experimental.pallas import tpu_sc as plsc

N = 50000
E = 800000
NSC = 2
NT = 16
HALF = 25000
ACC = 25088
RPT = 1568
WPT = 112
EPT = E // NT
ICH = 2000
NICH = EPT // ICH
CAPC = 392
CH = 128
SUP = 8


def _m8(x):
  return pl.multiple_of(x, 8)


_GDN = lax.GatherDimensionNumbers(
    offset_dims=(), collapsed_slice_dims=(0,), start_index_map=(0,))


def _prefix16(x, iota):
  for sh in (1, 2, 4, 8):
    idx = jnp.maximum(iota - sh, 0)
    shifted = lax.gather(x, idx[:, None], _GDN, slice_sizes=(1,),
                         mode=lax.GatherScatterMode.PROMISE_IN_BOUNDS)
    x = x + jnp.where(iota >= sh, shifted, 0)
  return x


def _fill_f32(ref, rows, value):
  v = jnp.full((16,), value, jnp.float32)
  def body(i, _):
    r = i >> 2
    c = (i & 3) * 16
    ref[r, pl.ds(c, 16)] = v
    return 0
  lax.fori_loop(0, rows * 4, body, 0)


def _prep_body(src_h, dst_h, deg_h, srcl_h, dstl_h, cnk_h,
               deg_sh, csrc, cdst, sbuf, dbuf, zbuf, ones_v, nbuf, sem):
  c = lax.axis_index("c")
  s = lax.axis_index("s")
  lo = c * HALF

  zero = jnp.zeros((16,), jnp.float32)
  def zfill(i, _):
    zbuf[pl.ds(i * 16, 16)] = zero
    return 0
  lax.fori_loop(0, RPT // 16, zfill, 0)
  pltpu.sync_copy(zbuf, deg_sh.at[pl.ds(_m8(s * RPT), RPT)])
  one = jnp.full((16,), 1.0, jnp.float32)
  for g in range(8):
    ones_v[pl.ds(g * 16, 16)] = one
  plsc.subcore_barrier()

  iota = lax.iota(jnp.int32, 16)
  ebase = s * EPT

  def chunk_body(j, off):
    pltpu.sync_copy(src_h.at[pl.ds(_m8(ebase + j * ICH), ICH)], sbuf)
    pltpu.sync_copy(dst_h.at[pl.ds(_m8(ebase + j * ICH), ICH)], dbuf)

    def grp(gi, off):
      sv = sbuf[pl.ds(gi * 16, 16)]
      dv = dbuf[pl.ds(gi * 16, 16)]
      m = (dv >= lo) & (dv < lo + HALF)
      dl = dv - lo
      pref = _prefix16(jnp.where(m, 1, 0).astype(jnp.int32), iota)
      pos = off + pref - 1
      plsc.store_scatter(csrc, [pos >> 7, pos & 127], sv, mask=m)
      plsc.store_scatter(cdst, [pos >> 7, pos & 127], dl, mask=m)
      return off + pref[15]

    return lax.fori_loop(0, ICH // 16, grp, off)

  off = lax.fori_loop(0, NICH, chunk_body, jnp.int32(0))

  pe = ((off + 1023) >> 10) << 10

  def pad_grp(t, _):
    pos = off + t * 16 + iota
    m = pos < pe
    dump = HALF + (pos & 63)
    spread = (pos * 61) & 16383
    plsc.store_scatter(cdst, [pos >> 7, pos & 127], dump, mask=m)
    plsc.store_scatter(csrc, [pos >> 7, pos & 127], spread, mask=m)
    return 0
  lax.fori_loop(0, 64, pad_grp, 0)

  n = pe >> 7

  def deg_body(jc, _):
    pltpu.sync_copy(ones_v, deg_sh.at[cdst.at[jc]], add=True)
    return 0
  lax.fori_loop(0, n, deg_body, 0)

  def wr_body(j8, _):
    pltpu.sync_copy(csrc.at[pl.ds(_m8(j8 * 8), 8)], srcl_h.at[c, s, pl.ds(_m8(j8 * 8), 8)])
    pltpu.sync_copy(cdst.at[pl.ds(_m8(j8 * 8), 8)], dstl_h.at[c, s, pl.ds(_m8(j8 * 8), 8)])
    return 0
  lax.fori_loop(0, (n + 7) >> 3, wr_body, 0)

  nbuf[...] = jnp.full((16,), n, jnp.int32)
  pltpu.sync_copy(nbuf, cnk_h.at[pl.ds(_m8((c * NT + s) * 16), 16)])

  plsc.subcore_barrier()
  pltpu.sync_copy(deg_sh.at[pl.ds(_m8(s * RPT), RPT)], zbuf)
  pltpu.sync_copy(zbuf, deg_h.at[pl.ds(_m8(c * ACC + s * RPT), RPT)])


@functools.partial(
    pl.kernel,
    out_type=(
        jax.ShapeDtypeStruct((NSC * ACC,), jnp.float32),
        jax.ShapeDtypeStruct((NSC, NT, CAPC, CH), jnp.int32),
        jax.ShapeDtypeStruct((NSC, NT, CAPC, CH), jnp.int32),
        jax.ShapeDtypeStruct((NSC * NT * 16,), jnp.int32),
    ),
    mesh=plsc.VectorSubcoreMesh(core_axis_name="c", subcore_axis_name="s"),
    compiler_params=pltpu.CompilerParams(needs_layout_passes=False, use_tc_tiling_on_sc=False),
    scratch_types=[
        pltpu.VMEM_SHARED((ACC,), jnp.float32),
        pltpu.VMEM((CAPC, CH), jnp.int32),
        pltpu.VMEM((CAPC, CH), jnp.int32),
        pltpu.VMEM((ICH,), jnp.int32),
        pltpu.VMEM((ICH,), jnp.int32),
        pltpu.VMEM((RPT,), jnp.float32),
        pltpu.VMEM((CH,), jnp.float32),
        pltpu.VMEM((16,), jnp.int32),
        pltpu.SemaphoreType.DMA,
    ],
)
def _sc_prep(src_h, dst_h, deg_h, srcl_h, dstl_h, cnk_h,
             deg_sh, csrc, cdst, sbuf, dbuf, zbuf, ones_v, nbuf, sem):
  _prep_body(src_h, dst_h, deg_h, srcl_h, dstl_h, cnk_h,
             deg_sh, csrc, cdst, sbuf, dbuf, zbuf, ones_v, nbuf, sem)


def _conv_body(g_h, srcl_h, dstl_h, cnk_h, acc_h,
               acc_sh, sidx, didx, rows, outb, nbuf, gsem, ssem):
  c = lax.axis_index("c")
  s = lax.axis_index("s")

  _fill_f32(outb, WPT, 0.0)
  for p in range(RPT // WPT):
    pltpu.sync_copy(outb, acc_sh.at[pl.ds(_m8(s * RPT + p * WPT), WPT)])
  plsc.subcore_barrier()

  pltpu.sync_copy(cnk_h.at[pl.ds(_m8((c * NT + s) * 16), 16)], nbuf)
  n = nbuf[...][0]

  def sup_body(j, _):
    pltpu.sync_copy(srcl_h.at[c, s, pl.ds(_m8(j * SUP), SUP)], sidx)
    pltpu.sync_copy(dstl_h.at[c, s, pl.ds(_m8(j * SUP), SUP)], didx)
    gd = [None] * SUP
    sd = [None] * SUP
    gd[0] = pltpu.async_copy(g_h.at[sidx.at[0]], rows.at[0], gsem)
    for k in range(SUP):
      gd[k].wait()
      sd[k] = pltpu.async_copy(rows.at[k & 1], acc_sh.at[didx.at[k]], ssem,
                               add=True)
      if k + 1 < SUP:
        if k >= 1:
          sd[k - 1].wait()
        gd[k + 1] = pltpu.async_copy(g_h.at[sidx.at[k + 1]],
                                     rows.at[(k + 1) & 1], gsem)
    sd[SUP - 2].wait()
    sd[SUP - 1].wait()
    return 0

  lax.fori_loop(0, n >> 3, sup_body, 0)
  plsc.subcore_barrier()

  for p in range(RPT // WPT):
    pltpu.sync_copy(acc_sh.at[pl.ds(_m8(s * RPT + p * WPT), WPT)], outb)
    pltpu.sync_copy(outb, acc_h.at[c, pl.ds(_m8(s * RPT + p * WPT), WPT)])


@functools.partial(
    pl.kernel,
    out_type=jax.ShapeDtypeStruct((NSC, ACC, 64), jnp.float32),
    mesh=plsc.VectorSubcoreMesh(core_axis_name="c", subcore_axis_name="s"),
    compiler_params=pltpu.CompilerParams(needs_layout_passes=False, use_tc_tiling_on_sc=False),
    scratch_types=[
        pltpu.VMEM_SHARED((ACC, 64), jnp.float32),
        pltpu.VMEM((SUP, CH), jnp.int32),
        pltpu.VMEM((SUP, CH), jnp.int32),
        pltpu.VMEM((2, CH, 64), jnp.float32),
        pltpu.VMEM((WPT, 64), jnp.float32),
        pltpu.VMEM((16,), jnp.int32),
        pltpu.SemaphoreType.DMA,
        pltpu.SemaphoreType.DMA,
    ],
)
def _sc_conv(g_h, srcl_h, dstl_h, cnk_h, acc_h,
             acc_sh, sidx, didx, rows, outb, nbuf, gsem, ssem):
  _conv_body(g_h, srcl_h, dstl_h, cnk_h, acc_h,
             acc_sh, sidx, didx, rows, outb, nbuf, gsem, ssem)


BN = 2000


def _k1_body(xb_r, xr_r, xt_r, deg_r, wn_r, bn_r, wr_r, br_r, wt_r, bt_r,
             wf_r, bf_r, wg1_r, g1_r):
  f32 = jnp.float32
  hb = jax.nn.relu(jnp.dot(xb_r[...], wn_r[...], preferred_element_type=f32) + bn_r[...])
  hr = jax.nn.relu(jnp.dot(xr_r[...], wr_r[...], preferred_element_type=f32) + br_r[...])
  ht = jax.nn.relu(jnp.dot(xt_r[...], wt_r[...], preferred_element_type=f32) + bt_r[...])
  wf = wf_r[...]
  pre = (jnp.dot(hb, wf[0:64], preferred_element_type=f32)
         + jnp.dot(hr, wf[64:128], preferred_element_type=f32)
         + jnp.dot(ht, wf[128:160], preferred_element_type=f32)
         + bf_r[...])
  h = jnp.where(pre >= 0, pre, 0.01 * pre)
  hw = jnp.dot(h, wg1_r[...], preferred_element_type=f32)
  dinv = lax.rsqrt(deg_r[...] + 1.0)
  g1_r[...] = hw * dinv


def _tc_k1(x_bbox, x_rf, x_txp, deg, Wn, bn, Wr, br, Wt, bt, Wf, bf, Wg1):
  full = lambda shape: pl.BlockSpec(shape, lambda i: (0, 0))
  row = lambda d: pl.BlockSpec((BN, d), lambda i: (i, 0))
  return pl.pallas_call(
      _k1_body,
      grid=(N // BN,),
      in_specs=[row(32), row(64), row(32), row(1),
                full((32, 64)), full((1, 64)), full((64, 64)), full((1, 64)),
                full((32, 32)), full((1, 32)), full((160, 128)), full((1, 128)),
                full((128, 64))],
      out_specs=row(64),
      out_shape=jax.ShapeDtypeStruct((N, 64), jnp.float32),
  )(x_bbox, x_rf, x_txp, deg, Wn, bn.reshape(1, -1), Wr, br.reshape(1, -1),
    Wt, bt.reshape(1, -1), Wf, bf.reshape(1, -1), Wg1)


def _k2_body(acc_r, g1_r, deg_r, bg1_r, wg2_r, h1_r, g2_r):
  f32 = jnp.float32
  dinv = lax.rsqrt(deg_r[...] + 1.0)
  h1 = jax.nn.relu(dinv * (acc_r[...] + g1_r[...]) + bg1_r[...])
  h1_r[...] = h1
  g2_r[...] = jnp.dot(h1, wg2_r[...], preferred_element_type=f32) * dinv


def _tc_k2(acc1, g1, deg, bg1, Wg2):
  full = lambda shape: pl.BlockSpec(shape, lambda i: (0, 0))
  row = lambda d: pl.BlockSpec((BN, d), lambda i: (i, 0))
  return pl.pallas_call(
      _k2_body,
      grid=(N // BN,),
      in_specs=[row(64), row(64), row(1), full((1, 64)), full((64, 64))],
      out_specs=[row(64), row(64)],
      out_shape=[jax.ShapeDtypeStruct((N, 64), jnp.float32),
                 jax.ShapeDtypeStruct((N, 64), jnp.float32)],
  )(acc1, g1, deg, bg1.reshape(1, -1), Wg2)


def _k3_body(acc_r, g2_r, deg_r, h1_r, bg2_r, ga_r, be_r, wo_r, bo_r, out_r):
  f32 = jnp.float32
  dinv = lax.rsqrt(deg_r[...] + 1.0)
  h2 = jax.nn.relu(dinv * (acc_r[...] + g2_r[...]) + bg2_r[...])
  scale = 1.0 / jnp.sqrt(jnp.float32(1.0 + 1e-5))
  ga = ga_r[...] * scale
  be = be_r[...]
  wo = wo_r[...]
  x1 = h1_r[...] * ga[:, 0:64] + be[:, 0:64]
  x2 = h2 * ga[:, 64:128] + be[:, 64:128]
  pre = (jnp.dot(x1, wo[0:64], preferred_element_type=f32)
         + jnp.dot(x2, wo[64:128], preferred_element_type=f32)
         + bo_r[...])
  out_r[...] = jax.nn.relu(pre)


def _tc_k3(acc2, g2, deg, h1, bg2, gamma, beta, Wo, bo):
  full = lambda shape: pl.BlockSpec(shape, lambda i: (0, 0))
  row = lambda d: pl.BlockSpec((BN, d), lambda i: (i, 0))
  return pl.pallas_call(
      _k3_body,
      grid=(N // BN,),
      in_specs=[row(64), row(64), row(1), row(64), full((1, 64)),
                full((1, 128)), full((1, 128)), full((128, 128)), full((1, 128))],
      out_specs=row(128),
      out_shape=jax.ShapeDtypeStruct((N, 128), jnp.float32),
  )(acc2, g2, deg, h1, bg2.reshape(1, -1), gamma.reshape(1, -1),
    beta.reshape(1, -1), Wo, bo.reshape(1, -1))


def _halves(x2):
  return jnp.concatenate([x2[0, :HALF], x2[1, :HALF]], axis=0)


@jax.jit
def kernel(x_bbox, x_rf, x_txp, edge_index, Wn, bn, Wr, br, Wt, bt, Wf, bf,
           Wg1, bg1, Wg2, bg2, gamma, beta, Wo, bo):
  src = edge_index[0]
  dst = edge_index[1]
  deg2, srcl, dstl, cnk = _sc_prep(src, dst)
  deg = _halves(deg2.reshape(NSC, ACC)).reshape(N, 1)
  g1 = _tc_k1(x_bbox, x_rf, x_txp, deg, Wn, bn, Wr, br, Wt, bt, Wf, bf, Wg1)
  acc1 = _halves(_sc_conv(g1, srcl, dstl, cnk))
  h1, g2 = _tc_k2(acc1, g1, deg, bg1, Wg2)
  acc2 = _halves(_sc_conv(g2, srcl, dstl, cnk))
  return _tc_k3(acc2, g2, deg, h1, bg2, gamma, beta, Wo, bo)


# speedup vs baseline: 20.4181x; 1.0190x over previous
import functools

import jax
import jax.numpy as jnp
from jax import lax
from jax.experimental import pallas as pl
from jax.experimental.pallas import tpu as pltpu
from jax.experimental.pallas import tpu_sc as plsc

N = 50000
E = 800000
NSC = 2
NT = 16
HALF = 25000
ACC = 25088
RPT = 1568
WPT = 112
EPT = E // NT
ICH = 2000
NICH = EPT // ICH
CAPC = 392
CH = 128
SUP = 8


def _m8(x):
  return pl.multiple_of(x, 8)


_GDN = lax.GatherDimensionNumbers(
    offset_dims=(), collapsed_slice_dims=(0,), start_index_map=(0,))


def _prefix16(x, iota):
  for sh in (1, 2, 4, 8):
    idx = jnp.maximum(iota - sh, 0)
    shifted = lax.gather(x, idx[:, None], _GDN, slice_sizes=(1,),
                         mode=lax.GatherScatterMode.PROMISE_IN_BOUNDS)
    x = x + jnp.where(iota >= sh, shifted, 0)
  return x


def _fill_f32(ref, rows, value):
  v = jnp.full((16,), value, jnp.float32)
  def body(i, _):
    r = i >> 2
    c = (i & 3) * 16
    ref[r, pl.ds(c, 16)] = v
    return 0
  lax.fori_loop(0, rows * 4, body, 0)


def _prep_body(src_h, dst_h, deg_h, srcl_h, dstl_h, cnk_h,
               deg_sh, csrc, cdst, sbuf, dbuf, zbuf, ones_v, nbuf, sem):
  c = lax.axis_index("c")
  s = lax.axis_index("s")
  lo = c * HALF

  zero = jnp.zeros((16,), jnp.float32)
  def zfill(i, _):
    zbuf[pl.ds(i * 16, 16)] = zero
    return 0
  lax.fori_loop(0, RPT // 16, zfill, 0)
  pltpu.sync_copy(zbuf, deg_sh.at[pl.ds(_m8(s * RPT), RPT)])
  one = jnp.full((16,), 1.0, jnp.float32)
  for g in range(8):
    ones_v[pl.ds(g * 16, 16)] = one
  plsc.subcore_barrier()

  iota = lax.iota(jnp.int32, 16)
  ebase = s * EPT

  def chunk_body(j, off):
    pltpu.sync_copy(src_h.at[pl.ds(_m8(ebase + j * ICH), ICH)], sbuf)
    pltpu.sync_copy(dst_h.at[pl.ds(_m8(ebase + j * ICH), ICH)], dbuf)

    def grp(gi, off):
      sv = sbuf[pl.ds(gi * 16, 16)]
      dv = dbuf[pl.ds(gi * 16, 16)]
      m = (dv >= lo) & (dv < lo + HALF)
      dl = dv - lo
      pref = _prefix16(jnp.where(m, 1, 0).astype(jnp.int32), iota)
      pos = off + pref - 1
      plsc.store_scatter(csrc, [pos >> 7, pos & 127], sv, mask=m)
      plsc.store_scatter(cdst, [pos >> 7, pos & 127], dl, mask=m)
      return off + pref[15]

    return lax.fori_loop(0, ICH // 16, grp, off)

  off = lax.fori_loop(0, NICH, chunk_body, jnp.int32(0))

  pe = ((off + 1023) >> 10) << 10

  def pad_grp(t, _):
    pos = off + t * 16 + iota
    m = pos < pe
    dump = HALF + (pos & 63)
    spread = (pos * 61) & 16383
    plsc.store_scatter(cdst, [pos >> 7, pos & 127], dump, mask=m)
    plsc.store_scatter(csrc, [pos >> 7, pos & 127], spread, mask=m)
    return 0
  lax.fori_loop(0, 64, pad_grp, 0)

  n = pe >> 7

  def deg_body(jc, _):
    pltpu.sync_copy(ones_v, deg_sh.at[cdst.at[jc]], add=True)
    return 0
  lax.fori_loop(0, n, deg_body, 0)

  def wr_body(j8, _):
    pltpu.sync_copy(csrc.at[pl.ds(_m8(j8 * 8), 8)], srcl_h.at[c, s, pl.ds(_m8(j8 * 8), 8)])
    pltpu.sync_copy(cdst.at[pl.ds(_m8(j8 * 8), 8)], dstl_h.at[c, s, pl.ds(_m8(j8 * 8), 8)])
    return 0
  lax.fori_loop(0, (n + 7) >> 3, wr_body, 0)

  nbuf[...] = jnp.full((16,), n, jnp.int32)
  pltpu.sync_copy(nbuf, cnk_h.at[pl.ds(_m8((c * NT + s) * 16), 16)])

  plsc.subcore_barrier()
  pltpu.sync_copy(deg_sh.at[pl.ds(_m8(s * RPT), RPT)], zbuf)
  pltpu.sync_copy(zbuf, deg_h.at[pl.ds(_m8(c * ACC + s * RPT), RPT)])


@functools.partial(
    pl.kernel,
    out_type=(
        jax.ShapeDtypeStruct((NSC * ACC,), jnp.float32),
        jax.ShapeDtypeStruct((NSC, NT, CAPC, CH), jnp.int32),
        jax.ShapeDtypeStruct((NSC, NT, CAPC, CH), jnp.int32),
        jax.ShapeDtypeStruct((NSC * NT * 16,), jnp.int32),
    ),
    mesh=plsc.VectorSubcoreMesh(core_axis_name="c", subcore_axis_name="s"),
    compiler_params=pltpu.CompilerParams(needs_layout_passes=False, use_tc_tiling_on_sc=False),
    scratch_types=[
        pltpu.VMEM_SHARED((ACC,), jnp.float32),
        pltpu.VMEM((CAPC, CH), jnp.int32),
        pltpu.VMEM((CAPC, CH), jnp.int32),
        pltpu.VMEM((ICH,), jnp.int32),
        pltpu.VMEM((ICH,), jnp.int32),
        pltpu.VMEM((RPT,), jnp.float32),
        pltpu.VMEM((CH,), jnp.float32),
        pltpu.VMEM((16,), jnp.int32),
        pltpu.SemaphoreType.DMA,
    ],
)
def _sc_prep(src_h, dst_h, deg_h, srcl_h, dstl_h, cnk_h,
             deg_sh, csrc, cdst, sbuf, dbuf, zbuf, ones_v, nbuf, sem):
  _prep_body(src_h, dst_h, deg_h, srcl_h, dstl_h, cnk_h,
             deg_sh, csrc, cdst, sbuf, dbuf, zbuf, ones_v, nbuf, sem)


def _conv_body(g_h, srcl_h, dstl_h, cnk_h, acc_h,
               acc_sh, sidx, didx, rows, outb, nbuf, gsem, ssem):
  c = lax.axis_index("c")
  s = lax.axis_index("s")

  _fill_f32(outb, WPT, 0.0)
  for p in range(RPT // WPT):
    pltpu.sync_copy(outb, acc_sh.at[pl.ds(_m8(s * RPT + p * WPT), WPT)])
  plsc.subcore_barrier()

  pltpu.sync_copy(cnk_h.at[pl.ds(_m8((c * NT + s) * 16), 16)], nbuf)
  n = nbuf[...][0]

  def sup_body(j, _):
    pltpu.sync_copy(srcl_h.at[c, s, pl.ds(_m8(j * SUP), SUP)], sidx)
    pltpu.sync_copy(dstl_h.at[c, s, pl.ds(_m8(j * SUP), SUP)], didx)
    gd = [None] * SUP
    sd = [None] * SUP
    gd[0] = pltpu.async_copy(g_h.at[sidx.at[0]], rows.at[0], gsem)
    for k in range(SUP):
      gd[k].wait()
      sd[k] = pltpu.async_copy(rows.at[k & 1], acc_sh.at[didx.at[k]], ssem,
                               add=True)
      if k + 1 < SUP:
        if k >= 1:
          sd[k - 1].wait()
        gd[k + 1] = pltpu.async_copy(g_h.at[sidx.at[k + 1]],
                                     rows.at[(k + 1) & 1], gsem)
    sd[SUP - 2].wait()
    sd[SUP - 1].wait()
    return 0

  lax.fori_loop(0, n >> 3, sup_body, 0)
  plsc.subcore_barrier()

  for p in range(RPT // WPT):
    pltpu.sync_copy(acc_sh.at[pl.ds(_m8(s * RPT + p * WPT), WPT)], outb)
    pltpu.sync_copy(outb, acc_h.at[c, pl.ds(_m8(s * RPT + p * WPT), WPT)])


@functools.partial(
    pl.kernel,
    out_type=jax.ShapeDtypeStruct((NSC, ACC, 64), jnp.float32),
    mesh=plsc.VectorSubcoreMesh(core_axis_name="c", subcore_axis_name="s"),
    compiler_params=pltpu.CompilerParams(needs_layout_passes=False, use_tc_tiling_on_sc=False),
    scratch_types=[
        pltpu.VMEM_SHARED((ACC, 64), jnp.float32),
        pltpu.VMEM((SUP, CH), jnp.int32),
        pltpu.VMEM((SUP, CH), jnp.int32),
        pltpu.VMEM((2, CH, 64), jnp.float32),
        pltpu.VMEM((WPT, 64), jnp.float32),
        pltpu.VMEM((16,), jnp.int32),
        pltpu.SemaphoreType.DMA,
        pltpu.SemaphoreType.DMA,
    ],
)
def _sc_conv(g_h, srcl_h, dstl_h, cnk_h, acc_h,
             acc_sh, sidx, didx, rows, outb, nbuf, gsem, ssem):
  _conv_body(g_h, srcl_h, dstl_h, cnk_h, acc_h,
             acc_sh, sidx, didx, rows, outb, nbuf, gsem, ssem)


BN = 1000


def _k1_body(xb_r, xr_r, xt_r, deg_r, wn_r, bn_r, wr_r, br_r, wt_r, bt_r,
             wf_r, bf_r, wg1_r, g1_r):
  f32 = jnp.float32
  deg = deg_r[...].reshape(BN, 1)
  hb = jax.nn.relu(jnp.dot(xb_r[...], wn_r[...], preferred_element_type=f32) + bn_r[...])
  hr = jax.nn.relu(jnp.dot(xr_r[...], wr_r[...], preferred_element_type=f32) + br_r[...])
  ht = jax.nn.relu(jnp.dot(xt_r[...], wt_r[...], preferred_element_type=f32) + bt_r[...])
  wf = wf_r[...]
  pre = (jnp.dot(hb, wf[0:64], preferred_element_type=f32)
         + jnp.dot(hr, wf[64:128], preferred_element_type=f32)
         + jnp.dot(ht, wf[128:160], preferred_element_type=f32)
         + bf_r[...])
  h = jnp.where(pre >= 0, pre, 0.01 * pre)
  hw = jnp.dot(h, wg1_r[...], preferred_element_type=f32)
  dinv = lax.rsqrt(deg + 1.0)
  g1_r[...] = hw * dinv


def _half_spec(d):
  return pl.BlockSpec((1, BN, d), lambda i: (i // 25, i % 25, 0))


def _tc_k1(x_bbox, x_rf, x_txp, deg3, Wn, bn, Wr, br, Wt, bt, Wf, bf, Wg1):
  full = lambda shape: pl.BlockSpec(shape, lambda i: (0, 0))
  row = lambda d: pl.BlockSpec((BN, d), lambda i: (i, 0))
  return pl.pallas_call(
      _k1_body,
      grid=(N // BN,),
      in_specs=[row(32), row(64), row(32), _half_spec(1),
                full((32, 64)), full((1, 64)), full((64, 64)), full((1, 64)),
                full((32, 32)), full((1, 32)), full((160, 128)), full((1, 128)),
                full((128, 64))],
      out_specs=row(64),
      out_shape=jax.ShapeDtypeStruct((N, 64), jnp.float32),
  )(x_bbox, x_rf, x_txp, deg3, Wn, bn.reshape(1, -1), Wr, br.reshape(1, -1),
    Wt, bt.reshape(1, -1), Wf, bf.reshape(1, -1), Wg1)


def _k2_body(acc_r, g1_r, deg_r, bg1_r, wg2_r, h1_r, g2_r):
  f32 = jnp.float32
  dinv = lax.rsqrt(deg_r[...].reshape(BN, 1) + 1.0)
  acc = acc_r[...].reshape(BN, 64)
  h1 = jax.nn.relu(dinv * (acc + g1_r[...]) + bg1_r[...])
  h1_r[...] = h1
  g2_r[...] = jnp.dot(h1, wg2_r[...], preferred_element_type=f32) * dinv


def _tc_k2(acc1h, g1, deg3, bg1, Wg2):
  full = lambda shape: pl.BlockSpec(shape, lambda i: (0, 0))
  row = lambda d: pl.BlockSpec((BN, d), lambda i: (i, 0))
  return pl.pallas_call(
      _k2_body,
      grid=(N // BN,),
      in_specs=[_half_spec(64), row(64), _half_spec(1), full((1, 64)),
                full((64, 64))],
      out_specs=[row(64), row(64)],
      out_shape=[jax.ShapeDtypeStruct((N, 64), jnp.float32),
                 jax.ShapeDtypeStruct((N, 64), jnp.float32)],
  )(acc1h, g1, deg3, bg1.reshape(1, -1), Wg2)


def _k3_body(acc_r, g2_r, deg_r, h1_r, bg2_r, ga_r, be_r, wo_r, bo_r, out_r):
  f32 = jnp.float32
  dinv = lax.rsqrt(deg_r[...].reshape(BN, 1) + 1.0)
  h2 = jax.nn.relu(dinv * (acc_r[...].reshape(BN, 64) + g2_r[...]) + bg2_r[...])
  scale = 1.0 / jnp.sqrt(jnp.float32(1.0 + 1e-5))
  ga = ga_r[...] * scale
  be = be_r[...]
  wo = wo_r[...]
  x1 = h1_r[...] * ga[:, 0:64] + be[:, 0:64]
  x2 = h2 * ga[:, 64:128] + be[:, 64:128]
  pre = (jnp.dot(x1, wo[0:64], preferred_element_type=f32)
         + jnp.dot(x2, wo[64:128], preferred_element_type=f32)
         + bo_r[...])
  out_r[...] = jax.nn.relu(pre)


def _tc_k3(acc2h, g2, deg3, h1, bg2, gamma, beta, Wo, bo):
  full = lambda shape: pl.BlockSpec(shape, lambda i: (0, 0))
  row = lambda d: pl.BlockSpec((BN, d), lambda i: (i, 0))
  return pl.pallas_call(
      _k3_body,
      grid=(N // BN,),
      in_specs=[_half_spec(64), row(64), _half_spec(1), row(64), full((1, 64)),
                full((1, 128)), full((1, 128)), full((128, 128)), full((1, 128))],
      out_specs=row(128),
      out_shape=jax.ShapeDtypeStruct((N, 128), jnp.float32),
  )(acc2h, g2, deg3, h1, bg2.reshape(1, -1), gamma.reshape(1, -1),
    beta.reshape(1, -1), Wo, bo.reshape(1, -1))


@jax.jit
def kernel(x_bbox, x_rf, x_txp, edge_index, Wn, bn, Wr, br, Wt, bt, Wf, bf,
           Wg1, bg1, Wg2, bg2, gamma, beta, Wo, bo):
  src = edge_index[0]
  dst = edge_index[1]
  deg2, srcl, dstl, cnk = _sc_prep(src, dst)
  deg3 = deg2.reshape(NSC, ACC, 1)
  g1 = _tc_k1(x_bbox, x_rf, x_txp, deg3, Wn, bn, Wr, br, Wt, bt, Wf, bf, Wg1)
  acc1h = _sc_conv(g1, srcl, dstl, cnk)
  h1, g2 = _tc_k2(acc1h, g1, deg3, bg1, Wg2)
  acc2h = _sc_conv(g2, srcl, dstl, cnk)
  return _tc_k3(acc2h, g2, deg3, h1, bg2, gamma, beta, Wo, bo)
